# speedup vs baseline: 1.5541x; 1.5541x over previous
import functools

import jax
import jax.numpy as jnp
from jax import lax
from jax.experimental import pallas as pl
from jax.experimental.pallas import tpu as pltpu
from jax.experimental.pallas import tpu_sc as plsc

_N = 100000
_D = 128
_NW = 32
_CHUNK = 128
_NCH = 26
_PW = _CHUNK * _NCH
_LAST = _N - _PW


def _make_kernel():
    mesh = plsc.VectorSubcoreMesh(core_axis_name="c", subcore_axis_name="s")

    @functools.partial(
        pl.kernel,
        mesh=mesh,
        out_type=jax.ShapeDtypeStruct((_N, _D), jnp.float32),
        scratch_types=[
            pltpu.VMEM((_PW,), jnp.int32),
            pltpu.VMEM((_CHUNK, _D), jnp.float32),
            pltpu.SemaphoreType.DMA,
        ],
    )
    def emb_kernel(z_hbm, table_hbm, out_hbm, idx_v, rows_v, sem):
        wid = lax.axis_index("s") * 2 + lax.axis_index("c")
        base = ((wid * _LAST) // (_NW - 1)) // 8 * 8
        pltpu.sync_copy(z_hbm.at[pl.ds(base, _PW)], idx_v)

        def body(j, carry):
            pltpu.async_copy(
                table_hbm.at[idx_v.at[pl.ds(j * _CHUNK, _CHUNK)]], rows_v, sem
            ).wait()
            pltpu.sync_copy(rows_v, out_hbm.at[pl.ds(base + j * _CHUNK, _CHUNK)])
            return carry

        lax.fori_loop(0, _NCH, body, 0)

    return emb_kernel


_emb = _make_kernel()


def kernel(Z, embeddings):
    return _emb(Z.astype(jnp.int32), embeddings)


# speedup vs baseline: 1.5661x; 1.0077x over previous
import functools

import jax
import jax.numpy as jnp
from jax import lax
from jax.experimental import pallas as pl
from jax.experimental.pallas import tpu as pltpu
from jax.experimental.pallas import tpu_sc as plsc

_N = 100000
_D = 128
_NW = 32
_CHUNK = 128
_NCH = 26
_PW = _CHUNK * _NCH
_LAST = _N - _PW


def _make_kernel():
    mesh = plsc.VectorSubcoreMesh(core_axis_name="c", subcore_axis_name="s")

    @functools.partial(
        pl.kernel,
        mesh=mesh,
        out_type=jax.ShapeDtypeStruct((_N, _D), jnp.float32),
        scratch_types=[
            pltpu.VMEM((_PW,), jnp.int32),
            pltpu.VMEM((2, _CHUNK, _D), jnp.float32),
            pltpu.SemaphoreType.DMA,
            pltpu.SemaphoreType.DMA,
            pltpu.SemaphoreType.DMA,
            pltpu.SemaphoreType.DMA,
        ],
    )
    def emb_kernel(z_hbm, table_hbm, out_hbm, idx_v, rows_v, sg0, sg1, sw0, sw1):
        wid = lax.axis_index("s") * 2 + lax.axis_index("c")
        base = ((wid * _LAST) // (_NW - 1)) // 8 * 8
        pltpu.sync_copy(z_hbm.at[pl.ds(base, _PW)], idx_v)
        sg = (sg0, sg1)
        sw = (sw0, sw1)

        def gather(c, b):
            return pltpu.make_async_copy(
                table_hbm.at[idx_v.at[pl.ds(c * _CHUNK, _CHUNK)]],
                rows_v.at[b],
                sg[b],
            )

        def write(c, b):
            return pltpu.make_async_copy(
                rows_v.at[b],
                out_hbm.at[pl.ds(base + c * _CHUNK, _CHUNK)],
                sw[b],
            )

        gather(0, 0).start()
        gather(1, 1).start()

        def body(i, carry):
            c0 = 2 * i
            for b in (0, 1):
                gather(c0 + b, b).wait()
                write(c0 + b, b).start()
            for b in (0, 1):
                write(c0 + b, b).wait()

                @pl.when(c0 + b + 2 < _NCH)
                def _():
                    gather(c0 + b + 2, b).start()

            return carry

        lax.fori_loop(0, _NCH // 2, body, 0)

    return emb_kernel


_emb = _make_kernel()


def kernel(Z, embeddings):
    return _emb(Z.astype(jnp.int32), embeddings)


# speedup vs baseline: 4.3313x; 2.7656x over previous
import functools

import jax
import jax.numpy as jnp
from jax import lax
from jax.experimental import pallas as pl
from jax.experimental.pallas import tpu as pltpu
from jax.experimental.pallas import tpu_sc as plsc

_N = 100000
_V = 119
_D = 128
_NW = 32
_CHUNK = 128
_SUB = 3
_SC_ROWS = _CHUNK * _SUB
_NSC = 9
_PW = _SC_ROWS * _NSC
_LAST = _N - _PW


def _make_kernel():
    mesh = plsc.VectorSubcoreMesh(core_axis_name="c", subcore_axis_name="s")

    @functools.partial(
        pl.kernel,
        mesh=mesh,
        out_type=jax.ShapeDtypeStruct((_N, _D), jnp.float32),
        scratch_types=[
            pltpu.VMEM_SHARED((_V, _D), jnp.float32),
            pltpu.VMEM((_PW,), jnp.int32),
            pltpu.VMEM((2, _SC_ROWS, _D), jnp.float32),
            pltpu.SemaphoreType.DMA,
            pltpu.SemaphoreType.DMA,
            pltpu.SemaphoreType.DMA,
            pltpu.SemaphoreType.DMA,
        ],
    )
    def emb_kernel(z_hbm, table_hbm, out_hbm, table_v, idx_v, rows_v,
                   sg0, sg1, sw0, sw1):
        wid = lax.axis_index("s") * 2 + lax.axis_index("c")
        base = ((wid * _LAST) // (_NW - 1)) // 8 * 8

        @pl.when(lax.axis_index("s") == 0)
        def _():
            pltpu.sync_copy(table_hbm, table_v)

        pltpu.sync_copy(z_hbm.at[pl.ds(base, _PW)], idx_v)
        plsc.subcore_barrier()
        sg = (sg0, sg1)
        sw = (sw0, sw1)

        def gathers(c, b):
            for u in range(_SUB):
                yield pltpu.make_async_copy(
                    table_v.at[idx_v.at[pl.ds((c * _SUB + u) * _CHUNK, _CHUNK)]],
                    rows_v.at[b, pl.ds(u * _CHUNK, _CHUNK)],
                    sg[b],
                )

        def fire_gathers(c, b):
            for g in gathers(c, b):
                g.start()

        def drain_gathers(c, b):
            for g in gathers(c, b):
                g.wait()

        def write(c, b):
            return pltpu.make_async_copy(
                rows_v.at[b],
                out_hbm.at[pl.ds(base + c * _SC_ROWS, _SC_ROWS)],
                sw[b],
            )

        fire_gathers(0, 0)
        fire_gathers(1, 1)

        def body(i, carry):
            c0 = 2 * i
            for b in (0, 1):

                @pl.when(c0 + b < _NSC)
                def _(b=b):
                    drain_gathers(c0 + b, b)
                    write(c0 + b, b).start()

            for b in (0, 1):

                @pl.when(c0 + b < _NSC)
                def _(b=b):
                    write(c0 + b, b).wait()

                    @pl.when(c0 + b + 2 < _NSC)
                    def _():
                        fire_gathers(c0 + b + 2, b)

            return carry

        lax.fori_loop(0, (_NSC + 1) // 2, body, 0)

    return emb_kernel


_emb = _make_kernel()


def kernel(Z, embeddings):
    return _emb(Z.astype(jnp.int32), embeddings)


# speedup vs baseline: 5.2251x; 1.2064x over previous
import functools

import jax
import jax.numpy as jnp
from jax import lax
from jax.experimental import pallas as pl
from jax.experimental.pallas import tpu as pltpu
from jax.experimental.pallas import tpu_sc as plsc

_N = 100000
_V = 119
_D = 128
_NW = 32
_CHUNK = 128
_SUB = 3
_SC_ROWS = _CHUNK * _SUB
_NSC = 9
_PW = _SC_ROWS * _NSC
_LAST = _N - _PW


def _make_kernel():
    mesh = plsc.VectorSubcoreMesh(core_axis_name="c", subcore_axis_name="s")

    @functools.partial(
        pl.kernel,
        mesh=mesh,
        out_type=jax.ShapeDtypeStruct((_N, _D), jnp.float32),
        scratch_types=[
            pltpu.VMEM_SHARED((_V, _D), jnp.float32),
            pltpu.VMEM((_PW,), jnp.int32),
            pltpu.VMEM((2, _SC_ROWS, _D), jnp.float32),
            pltpu.SemaphoreType.DMA,
            pltpu.SemaphoreType.DMA,
            pltpu.SemaphoreType.DMA,
            pltpu.SemaphoreType.DMA,
        ],
    )
    def emb_kernel(z_hbm, table_hbm, out_hbm, table_sh, idx_v, rows_v,
                   sg0, sg1, sw0, sw1):
        wid = lax.axis_index("s") * 2 + lax.axis_index("c")
        base = ((wid * _LAST) // (_NW - 1)) // 8 * 8

        @pl.when(lax.axis_index("s") == 0)
        def _():
            pltpu.sync_copy(table_hbm, table_sh)

        pltpu.sync_copy(z_hbm.at[pl.ds(base, _PW)], idx_v)
        plsc.subcore_barrier()
        sg = (sg0, sg1)
        sw = (sw0, sw1)

        def gathers(c, b):
            for u in range(_SUB):
                yield pltpu.make_async_copy(
                    table_sh.at[idx_v.at[pl.ds((c * _SUB + u) * _CHUNK, _CHUNK)]],
                    rows_v.at[b, pl.ds(u * _CHUNK, _CHUNK)],
                    sg[b],
                )

        def write(c, b):
            return pltpu.make_async_copy(
                rows_v.at[b],
                out_hbm.at[pl.ds(base + c * _SC_ROWS, _SC_ROWS)],
                sw[b],
            )

        def chunk_step(c, b):
            for g in gathers(c, b):
                g.wait()
            write(c, b).start()

            @pl.when(c > 0)
            def _():
                write(c - 1, 1 - b).wait()

            @pl.when(c + 1 < _NSC)
            def _():
                for g in gathers(c + 1, 1 - b):
                    g.start()

        for g in gathers(0, 0):
            g.start()

        def body(i, carry):
            for b in (0, 1):
                c = 2 * i + b

                @pl.when(c < _NSC)
                def _(c=c, b=b):
                    chunk_step(c, b)

            return carry

        lax.fori_loop(0, (_NSC + 1) // 2, body, 0)
        write(_NSC - 1, (_NSC - 1) % 2).wait()

    return emb_kernel


_emb = _make_kernel()


def kernel(Z, embeddings):
    return _emb(Z.astype(jnp.int32), embeddings)
